# Initial kernel scaffold; baseline (speedup 1.0000x reference)
import jax
import jax.numpy as jnp
from jax.experimental import pallas as pl


def kernel(x, W):
    raise NotImplementedError("write your pallas kernel here")


# speedup vs baseline: 11.6035x; 11.6035x over previous
import functools

import jax
import jax.numpy as jnp
import numpy as np
from jax import lax
from jax.experimental import pallas as pl
from jax.experimental.pallas import tpu as pltpu
from jax.experimental.pallas import tpu_sc as plsc

F = 26
FD = 4000
D = 16
B = 4096
TBL = F * FD
V = F * TBL
FF = F * F

NC = 2
NS = 16
NW = NC * NS
BPW = B // NW
CB = 4
NCHUNK = BPW // CB
ROWS = CB * FF
SUB = 104
NSUB = ROWS // SUB

_PI, _PJ = np.triu_indices(F, k=1)
_OFF_A = [int(i) * F + int(j) for i, j in zip(_PI, _PJ)]
_OFF_B = [int(j) * F + int(i) for i, j in zip(_PI, _PJ)]


def _sc_body(w_hbm, idx_hbm, out_hbm, idx_v, rows_v, res_v, out_v, sem):
    cid = lax.axis_index("c")
    sid = lax.axis_index("s")
    wid = sid * NC + cid
    b0 = wid * BPW

    def chunk_body(c, carry):
        start = (b0 + c * CB) * FF
        pltpu.sync_copy(idx_hbm.at[pl.ds(start, ROWS)], idx_v)
        handles = []
        for s in range(NSUB):
            handles.append(
                pltpu.async_copy(
                    w_hbm.at[idx_v.at[pl.ds(s * SUB, SUB)]],
                    rows_v.at[pl.ds(s * SUB, SUB)],
                    sem,
                )
            )
        for h in handles:
            h.wait()

        def b_body(bl, carry2):
            rb = bl * FF
            acc = jnp.zeros((D,), jnp.float32)
            for oa, ob in zip(_OFF_A, _OFF_B):
                acc = acc + rows_v[rb + oa] * rows_v[rb + ob]
            res_v[c * CB + bl] = acc
            return carry2

        lax.fori_loop(0, CB, b_body, 0, unroll=False)
        return carry

    lax.fori_loop(0, NCHUNK, chunk_body, 0, unroll=False)

    lanes = lax.iota(jnp.int32, D)
    for grp in range(BPW // D):
        base = grp * D
        tot = jnp.zeros((D,), jnp.float32)
        for l in range(D):
            s = jnp.sum(res_v[base + l])
            tot = jnp.where(lanes == l, s, tot)
        out_v[pl.ds(base, D)] = tot

    pltpu.sync_copy(out_v, out_hbm.at[pl.ds(b0, BPW)])


@functools.cache
def _ffm_kernel():
    return pl.kernel(
        _sc_body,
        out_type=jax.ShapeDtypeStruct((B,), jnp.float32),
        mesh=plsc.VectorSubcoreMesh(
            core_axis_name="c", subcore_axis_name="s",
            num_cores=NC, num_subcores=NS,
        ),
        compiler_params=pltpu.CompilerParams(
            needs_layout_passes=False, use_tc_tiling_on_sc=False
        ),
        scratch_types=[
            pltpu.VMEM((ROWS,), jnp.int32),
            pltpu.VMEM((ROWS, D), jnp.float32),
            pltpu.VMEM((BPW, D), jnp.float32),
            pltpu.VMEM((BPW,), jnp.float32),
            pltpu.SemaphoreType.DMA,
        ],
    )


@jax.jit
def kernel(x, W):
    wflat = W.reshape(V, D)
    g = x.astype(jnp.int32) + (jnp.arange(F, dtype=jnp.int32) * FD)[None, :]
    t_off = jnp.arange(F, dtype=jnp.int32) * TBL
    idx = g[:, :, None] + t_off[None, None, :]
    return _ffm_kernel()(wflat, idx.reshape(B * FF))


# speedup vs baseline: 11.8338x; 1.0198x over previous
import functools

import jax
import jax.numpy as jnp
import numpy as np
from jax import lax
from jax.experimental import pallas as pl
from jax.experimental.pallas import tpu as pltpu
from jax.experimental.pallas import tpu_sc as plsc

F = 26
FD = 4000
D = 16
B = 4096
TBL = F * FD
V = F * TBL
FF = F * F

NC = 2
NS = 16
NW = NC * NS
BPW = B // NW
CB = 4
NCHUNK = BPW // CB
ROWS = CB * FF
SUB = 104
NSUB = ROWS // SUB

_PI, _PJ = np.triu_indices(F, k=1)
_OFF_A = [int(j) * F + int(i) for i, j in zip(_PI, _PJ)]
_OFF_B = [int(i) * F + int(j) for i, j in zip(_PI, _PJ)]


def _sc_body(w_hbm, x_hbm, out_hbm, x_v, idx_v, rows_v, res_v, out_v, sem):
    cid = lax.axis_index("c")
    sid = lax.axis_index("s")
    wid = sid * NC + cid
    b0 = wid * BPW

    pltpu.sync_copy(x_hbm.at[pl.ds(b0 * F, BPW * F)], x_v.at[pl.ds(0, BPW * F)])

    lanes = lax.iota(jnp.int32, D)
    f_lo = lanes * FD
    f_hi = (lanes + D) * FD

    def chunk_body(c, carry):
        def bl_body(bl, carry2):
            xoff = (c * CB + bl) * F
            vlo = x_v[pl.ds(xoff, D)] + f_lo
            vhi = x_v[pl.ds(xoff + D, D)] + f_hi

            def t_body(t, carry3):
                off = bl * FF + t * F
                tb = t * TBL
                idx_v[pl.ds(off, D)] = vlo + tb
                idx_v[pl.ds(off + D, D)] = vhi + tb
                return carry3

            lax.fori_loop(0, F, t_body, 0, unroll=False)
            return carry2

        lax.fori_loop(0, CB, bl_body, 0, unroll=False)

        handles = []
        for s in range(NSUB):
            handles.append(
                pltpu.async_copy(
                    w_hbm.at[idx_v.at[pl.ds(s * SUB, SUB)]],
                    rows_v.at[pl.ds(s * SUB, SUB)],
                    sem,
                )
            )
        for h in handles:
            h.wait()

        def b_body(bl, carry2):
            rb = bl * FF
            acc = jnp.zeros((D,), jnp.float32)
            for oa, ob in zip(_OFF_A, _OFF_B):
                acc = acc + rows_v[rb + oa] * rows_v[rb + ob]
            res_v[c * CB + bl] = acc
            return carry2

        lax.fori_loop(0, CB, b_body, 0, unroll=False)
        return carry

    lax.fori_loop(0, NCHUNK, chunk_body, 0, unroll=False)

    for grp in range(BPW // D):
        base = grp * D
        tot = jnp.zeros((D,), jnp.float32)
        for l in range(D):
            s = jnp.sum(res_v[base + l])
            tot = jnp.where(lanes == l, s, tot)
        out_v[pl.ds(base, D)] = tot

    pltpu.sync_copy(out_v, out_hbm.at[pl.ds(b0, BPW)])


@functools.cache
def _ffm_kernel():
    return pl.kernel(
        _sc_body,
        out_type=jax.ShapeDtypeStruct((B,), jnp.float32),
        mesh=plsc.VectorSubcoreMesh(
            core_axis_name="c", subcore_axis_name="s",
            num_cores=NC, num_subcores=NS,
        ),
        compiler_params=pltpu.CompilerParams(
            needs_layout_passes=False, use_tc_tiling_on_sc=False
        ),
        scratch_types=[
            pltpu.VMEM((BPW * F + D,), jnp.int32),
            pltpu.VMEM((ROWS + D,), jnp.int32),
            pltpu.VMEM((ROWS, D), jnp.float32),
            pltpu.VMEM((BPW, D), jnp.float32),
            pltpu.VMEM((BPW,), jnp.float32),
            pltpu.SemaphoreType.DMA,
        ],
    )


@jax.jit
def kernel(x, W):
    wflat = W.reshape(V, D)
    xflat = x.astype(jnp.int32).reshape(B * F)
    return _ffm_kernel()(wflat, xflat)


# speedup vs baseline: 11.9361x; 1.0086x over previous
import functools

import jax
import jax.numpy as jnp
import numpy as np
from jax import lax
from jax.experimental import pallas as pl
from jax.experimental.pallas import tpu as pltpu
from jax.experimental.pallas import tpu_sc as plsc

F = 26
FD = 4000
D = 16
B = 4096
TBL = F * FD
FF = F * F

NC = 2
NS = 16
NW = NC * NS
BPW = B // NW
CB = 4
NCHUNK = BPW // CB
SUB = CB * F
ROWS = F * SUB

_PI, _PJ = np.triu_indices(F, k=1)
_OFF_A = [int(j) * SUB + int(i) for i, j in zip(_PI, _PJ)]
_OFF_B = [int(i) * SUB + int(j) for i, j in zip(_PI, _PJ)]


def _sc_body(w_hbm, x_hbm, out_hbm, x_v, idx_v, rows_v, res_v, out_v, sem):
    cid = lax.axis_index("c")
    sid = lax.axis_index("s")
    wid = sid * NC + cid
    b0 = wid * BPW

    pltpu.sync_copy(x_hbm.at[pl.ds(b0 * F, BPW * F)], x_v.at[pl.ds(0, BPW * F)])

    lanes = lax.iota(jnp.int32, D)
    f_lo = lanes * FD
    f_hi = (lanes + D) * FD

    def chunk_body(c, carry):
        def bl_body(bl, carry2):
            xoff = (c * CB + bl) * F
            off = bl * F
            idx_v[pl.ds(off, D)] = x_v[pl.ds(xoff, D)] + f_lo
            idx_v[pl.ds(off + D, D)] = x_v[pl.ds(xoff + D, D)] + f_hi
            return carry2

        lax.fori_loop(0, CB, bl_body, 0, unroll=False)

        handles = []
        for t in range(F):
            handles.append(
                pltpu.async_copy(
                    w_hbm.at[t].at[idx_v.at[pl.ds(0, SUB)]],
                    rows_v.at[pl.ds(t * SUB, SUB)],
                    sem,
                )
            )
        for h in handles:
            h.wait()

        def b_body(bl, carry2):
            rb = bl * F
            acc = jnp.zeros((D,), jnp.float32)
            for oa, ob in zip(_OFF_A, _OFF_B):
                acc = acc + rows_v[rb + oa] * rows_v[rb + ob]
            res_v[c * CB + bl] = acc
            return carry2

        lax.fori_loop(0, CB, b_body, 0, unroll=False)
        return carry

    lax.fori_loop(0, NCHUNK, chunk_body, 0, unroll=False)

    for grp in range(BPW // D):
        base = grp * D
        tot = jnp.zeros((D,), jnp.float32)
        for l in range(D):
            s = jnp.sum(res_v[base + l])
            tot = jnp.where(lanes == l, s, tot)
        out_v[pl.ds(base, D)] = tot

    pltpu.sync_copy(out_v, out_hbm.at[pl.ds(b0, BPW)])


@functools.cache
def _ffm_kernel():
    return pl.kernel(
        _sc_body,
        out_type=jax.ShapeDtypeStruct((B,), jnp.float32),
        mesh=plsc.VectorSubcoreMesh(
            core_axis_name="c", subcore_axis_name="s",
            num_cores=NC, num_subcores=NS,
        ),
        compiler_params=pltpu.CompilerParams(
            needs_layout_passes=False, use_tc_tiling_on_sc=False
        ),
        scratch_types=[
            pltpu.VMEM((BPW * F + D,), jnp.int32),
            pltpu.VMEM((SUB + D,), jnp.int32),
            pltpu.VMEM((ROWS, D), jnp.float32),
            pltpu.VMEM((BPW, D), jnp.float32),
            pltpu.VMEM((BPW,), jnp.float32),
            pltpu.SemaphoreType.DMA,
        ],
    )


@jax.jit
def kernel(x, W):
    xflat = x.astype(jnp.int32).reshape(B * F)
    return _ffm_kernel()(W, xflat)


# speedup vs baseline: 11.9395x; 1.0003x over previous
import functools

import jax
import jax.numpy as jnp
import numpy as np
from jax import lax
from jax.experimental import pallas as pl
from jax.experimental.pallas import tpu as pltpu
from jax.experimental.pallas import tpu_sc as plsc

F = 26
FD = 4000
D = 16
B = 4096
TBL = F * FD
FF = F * F

NC = 2
NS = 16
NW = NC * NS
BPW = B // NW
CB = 4
NCHUNK = BPW // CB
SUB = CB * F
ROWS = F * SUB

_PI, _PJ = np.triu_indices(F, k=1)
_OFF_A = [int(j) * SUB + int(i) for i, j in zip(_PI, _PJ)]
_OFF_B = [int(i) * SUB + int(j) for i, j in zip(_PI, _PJ)]


def _sc_body(w_hbm, x_hbm, out_hbm, x_v, idx_v, rows_v, res_v, out_v, sem):
    cid = lax.axis_index("c")
    sid = lax.axis_index("s")
    wid = sid * NC + cid
    b0 = wid * BPW

    pltpu.sync_copy(x_hbm.at[pl.ds(b0 * F, BPW * F)], x_v.at[pl.ds(0, BPW * F)])

    lanes = lax.iota(jnp.int32, D)
    f_lo = lanes * FD
    f_hi = (lanes + D) * FD

    def chunk_body(c, carry):
        def bl_body(bl, carry2):
            xoff = (c * CB + bl) * F
            off = bl * F
            idx_v[pl.ds(off, D)] = x_v[pl.ds(xoff, D)] + f_lo
            idx_v[pl.ds(off + D, D)] = x_v[pl.ds(xoff + D, D)] + f_hi
            return carry2

        lax.fori_loop(0, CB, bl_body, 0, unroll=False)

        handles = []
        for t in range(F):
            handles.append(
                pltpu.async_copy(
                    w_hbm.at[t].at[idx_v.at[pl.ds(0, SUB)]],
                    rows_v.at[pl.ds(t * SUB, SUB)],
                    sem,
                )
            )
        for h in handles:
            h.wait()

        def b_body(bl, carry2):
            rb = bl * F
            acc = jnp.zeros((D,), jnp.float32)
            for oa, ob in zip(_OFF_A, _OFF_B):
                acc = acc + rows_v[rb + oa] * rows_v[rb + ob]
            res_v[c * CB + bl] = acc
            return carry2

        lax.fori_loop(0, CB, b_body, 0, unroll=False)
        return carry

    lax.fori_loop(0, NCHUNK, chunk_body, 0, unroll=False)

    for grp in range(BPW // D):
        base = grp * D
        tot = jnp.zeros((D,), jnp.float32)
        for l in range(D):
            s = jnp.sum(res_v[base + l])
            tot = jnp.where(lanes == l, s, tot)
        out_v[pl.ds(base, D)] = tot

    pltpu.sync_copy(out_v, out_hbm.at[pl.ds(b0, BPW)])


@functools.cache
def _ffm_kernel():
    return pl.kernel(
        _sc_body,
        out_type=jax.ShapeDtypeStruct((B,), jnp.float32),
        mesh=plsc.VectorSubcoreMesh(
            core_axis_name="c", subcore_axis_name="s",
            num_cores=NC, num_subcores=NS,
        ),
        compiler_params=pltpu.CompilerParams(
            needs_layout_passes=False, use_tc_tiling_on_sc=False
        ),
        scratch_types=[
            pltpu.VMEM((BPW * F + D,), jnp.int32),
            pltpu.VMEM((SUB + D,), jnp.int32),
            pltpu.VMEM((ROWS, D), jnp.float32),
            pltpu.VMEM((BPW, D), jnp.float32),
            pltpu.VMEM((BPW,), jnp.float32),
            pltpu.SemaphoreType.DMA,
        ],
    )


@jax.jit
def kernel(x, W):
    xflat = x.astype(jnp.int32).reshape(B * F)
    wsc = W.reshape(F * TBL * D).reshape(F, TBL, D)
    return _ffm_kernel()(wsc, xflat)
